# baseline (speedup 1.0000x reference)
import jax
import jax.numpy as jnp
from jax import lax
from jax.experimental import pallas as pl
from jax.experimental.pallas import tpu as pltpu
from jax.experimental.pallas import tpu_sc as plsc

N = 10000
E = 160000
D = 256
H = 256
G = 64

NCORES = 2
NSUB = 16
NW = NCORES * NSUB

NPAD = 10240
NHALVES = 2
NRANGE = NPAD // NHALVES
RPT = NRANGE // NW
ACCR = RPT + 8
EPAD2 = E + 128

CHUNK = 64
LG = 16

BN = 400
NB = N // BN


def _sc_body(half, h_hbm, src_hbm, dst_hbm, blo_hbm, bhi_hbm, s_out, deg_out,
             sidx_v, didx_v, rows_v, bnd_v, acc_v, dacc_v, sem):
    sc = lax.axis_index("c")
    t = lax.axis_index("s")
    wid = sc * NSUB + t
    lanes = lax.broadcasted_iota(jnp.int32, (LG,), 0)
    zf = jnp.zeros((LG,), jnp.float32)
    ones = jnp.full((LG,), 1.0, jnp.float32)

    def zero_body(r, carry):
        for j in range(H // LG):
            acc_v[r, pl.ds(LG * j, LG)] = zf
        dacc_v[r] = zf
        return carry

    lax.fori_loop(0, ACCR, zero_body, 0)

    def pick(vec, i):
        s = jnp.int32(0)
        for lane in range(LG):
            s = jnp.where(i == lane, vec[lane], s)
        return s

    pltpu.sync_copy(blo_hbm, bnd_v)
    blo_s = pick(bnd_v[pl.ds(sc * LG, LG)], t)
    pltpu.sync_copy(bhi_hbm, bnd_v)
    bhi_s = pick(bnd_v[pl.ds(sc * LG, LG)], t)
    clo = (blo_s >> 3) << 3
    nch = (bhi_s - clo + (CHUNK - 1)) >> 6
    lo_node = half * NRANGE + wid * RPT

    def chunk_body(c, carry):
        e0 = pl.multiple_of(clo + c * CHUNK, 8)
        pltpu.sync_copy(src_hbm.at[pl.ds(e0, CHUNK)], sidx_v)
        pltpu.sync_copy(dst_hbm.at[pl.ds(e0, CHUNK)], didx_v)
        pltpu.async_copy(h_hbm.at[sidx_v], rows_v, sem).wait()
        for g in range(CHUNK // LG):
            d = didx_v[pl.ds(g * LG, LG)]
            ld = d - lo_node
            ld = jnp.where((ld >= 0) & (ld < RPT), ld, RPT)
            for lane in range(LG):
                ld_s = ld[lane]
                e = g * LG + lane
                for j in range(H // LG):
                    v = rows_v[e, pl.ds(LG * j, LG)]
                    plsc.addupdate(acc_v.at[ld_s, pl.ds(LG * j, LG)], v)
                plsc.addupdate(dacc_v.at[ld_s], ones)
        return carry

    lax.fori_loop(0, nch, chunk_body, 0)

    out_r = wid * RPT
    pltpu.sync_copy(acc_v.at[pl.ds(0, RPT)], s_out.at[pl.ds(out_r, RPT)])
    pltpu.sync_copy(dacc_v.at[pl.ds(0, RPT)], deg_out.at[pl.ds(out_r, RPT)])


def _sc_segment_sum(h, src_s, dst_s, blo, bhi, half):
    import functools
    mesh = plsc.VectorSubcoreMesh(core_axis_name="c", subcore_axis_name="s")
    k = pl.kernel(
        functools.partial(_sc_body, half),
        out_type=[
            jax.ShapeDtypeStruct((NRANGE, H), jnp.float32),
            jax.ShapeDtypeStruct((NRANGE, LG), jnp.float32),
        ],
        mesh=mesh,
        scratch_types=[
            pltpu.VMEM((CHUNK,), jnp.int32),
            pltpu.VMEM((CHUNK,), jnp.int32),
            pltpu.VMEM((CHUNK, H), jnp.float32),
            pltpu.VMEM((NW,), jnp.int32),
            pltpu.VMEM((ACCR, H), jnp.float32),
            pltpu.VMEM((ACCR, LG), jnp.float32),
            pltpu.SemaphoreType.DMA,
        ],
    )
    return k(h, src_s, dst_s, blo, bhi)


def _ln(acc, g, b):
    mu = jnp.mean(acc, axis=-1, keepdims=True)
    var = jnp.mean((acc - mu) ** 2, axis=-1, keepdims=True)
    return (acc - mu) * lax.rsqrt(var + 1e-5) * g + b


def _fuse_body(xt_ref, xv_ref, twt_ref, vwt_ref, b_ref, g_ref, bb_ref, o_ref):
    acc = lax.dot_general(xt_ref[...], twt_ref[...], (((1,), (0,)), ((), ())),
                          preferred_element_type=jnp.float32)
    acc = acc + lax.dot_general(xv_ref[...], vwt_ref[...],
                                (((1,), (0,)), ((), ())),
                                preferred_element_type=jnp.float32)
    acc = acc + b_ref[...]
    o_ref[...] = _ln(acc, g_ref[...], bb_ref[...])


def _fuse(xt, xv, twt, vwt, b, g, bb):
    return pl.pallas_call(
        _fuse_body,
        grid=(NB,),
        in_specs=[
            pl.BlockSpec((BN, D), lambda i: (i, 0)),
            pl.BlockSpec((BN, D), lambda i: (i, 0)),
            pl.BlockSpec((D, H), lambda i: (0, 0)),
            pl.BlockSpec((D, H), lambda i: (0, 0)),
            pl.BlockSpec((1, H), lambda i: (0, 0)),
            pl.BlockSpec((1, H), lambda i: (0, 0)),
            pl.BlockSpec((1, H), lambda i: (0, 0)),
        ],
        out_specs=pl.BlockSpec((BN, H), lambda i: (i, 0)),
        out_shape=jax.ShapeDtypeStruct((N, H), jnp.float32),
    )(xt, xv, twt, vwt, b, g, bb)


def _layer_body(h_ref, s_ref, d_ref, mwt_ref, swt_ref,
                b_ref, g_ref, bb_ref, o_ref):
    h = h_ref[...]
    dinv = 1.0 / jnp.maximum(d_ref[...][:, 0:1], 1.0)
    agg = s_ref[...] * dinv
    acc = lax.dot_general(h, swt_ref[...], (((1,), (0,)), ((), ())),
                          preferred_element_type=jnp.float32)
    acc = acc + lax.dot_general(agg, mwt_ref[...], (((1,), (0,)), ((), ())),
                                preferred_element_type=jnp.float32)
    acc = jnp.maximum(acc + b_ref[...], 0.0)
    o_ref[...] = h + _ln(acc, g_ref[...], bb_ref[...])


def _layer(h, s, d, mwt, swt, b, g, bb):
    return pl.pallas_call(
        _layer_body,
        grid=(NB,),
        in_specs=[
            pl.BlockSpec((BN, H), lambda i: (i, 0)),
            pl.BlockSpec((BN, H), lambda i: (i, 0)),
            pl.BlockSpec((BN, LG), lambda i: (i, 0)),
            pl.BlockSpec((H, H), lambda i: (0, 0)),
            pl.BlockSpec((H, H), lambda i: (0, 0)),
            pl.BlockSpec((1, H), lambda i: (0, 0)),
            pl.BlockSpec((1, H), lambda i: (0, 0)),
            pl.BlockSpec((1, H), lambda i: (0, 0)),
        ],
        out_specs=pl.BlockSpec((BN, H), lambda i: (i, 0)),
        out_shape=jax.ShapeDtypeStruct((N, H), jnp.float32),
    )(h, s, d, mwt, swt, b, g, bb)


def _pool_body(batch_ref, h_ref, h1wt_ref, h1b_ref, h2w_ref, h2b_ref, o_ref,
               sums_acc, cnt_acc):
    i = pl.program_id(0)

    @pl.when(i == 0)
    def _():
        sums_acc[...] = jnp.zeros_like(sums_acc)
        cnt_acc[...] = jnp.zeros_like(cnt_acc)

    b = batch_ref[0]
    gid = lax.broadcasted_iota(jnp.int32, (G, BN), 0)
    onehot = (gid == jnp.broadcast_to(b, (G, BN))).astype(jnp.float32)
    sums_acc[...] += lax.dot_general(onehot, h_ref[...],
                                     (((1,), (0,)), ((), ())),
                                     preferred_element_type=jnp.float32)
    cnt_acc[...] += lax.dot_general(onehot, jnp.ones((BN, H), jnp.float32),
                                    (((1,), (0,)), ((), ())),
                                    preferred_element_type=jnp.float32)

    @pl.when(i == NB - 1)
    def _():
        gmean = sums_acc[...] * (1.0 / jnp.maximum(cnt_acc[...], 1.0))
        z = lax.dot_general(gmean, h1wt_ref[...], (((1,), (0,)), ((), ())),
                            preferred_element_type=jnp.float32)
        z = jnp.maximum(z + h1b_ref[...], 0.0)
        lg = lax.dot_general(z, h2w_ref[...], (((1,), (1,)), ((), ())),
                             preferred_element_type=jnp.float32)
        o_ref[...] = lg + h2b_ref[0, 0]


def _pool(batch_r, h, h1wt, h1b, h2w, h2b):
    return pl.pallas_call(
        _pool_body,
        grid=(NB,),
        in_specs=[
            pl.BlockSpec((1, 1, BN), lambda i: (i, 0, 0)),
            pl.BlockSpec((BN, H), lambda i: (i, 0)),
            pl.BlockSpec((H, H), lambda i: (0, 0)),
            pl.BlockSpec((1, H), lambda i: (0, 0)),
            pl.BlockSpec((128, H), lambda i: (0, 0)),
            pl.BlockSpec((1, 1), lambda i: (0, 0)),
        ],
        out_specs=pl.BlockSpec((G, 128), lambda i: (0, 0)),
        out_shape=jax.ShapeDtypeStruct((G, 128), jnp.float32),
        scratch_shapes=[
            pltpu.VMEM((G, H), jnp.float32),
            pltpu.VMEM((G, H), jnp.float32),
        ],
    )(batch_r, h, h1wt, h1b, h2w, h2b)


def kernel(x_text, x_vis, tp_w, tp_b, vp_w, vp_b, fln_g, fln_b,
           l0_msg_w, l0_self_w, l0_self_b, l0_ln_g, l0_ln_b,
           l1_msg_w, l1_self_w, l1_self_b, l1_ln_g, l1_ln_b,
           l2_msg_w, l2_self_w, l2_self_b, l2_ln_g, l2_ln_b,
           h1_w, h1_b, h2_w, h2_b, edge_index, batch):
    src = edge_index[0]
    dst = edge_index[1]
    src_p = jnp.concatenate([src, jnp.zeros((EPAD2 - E,), jnp.int32)])
    dst_p = jnp.concatenate([dst, jnp.full((EPAD2 - E,), NPAD, jnp.int32)])
    perm = jnp.argsort(dst_p)
    src_s = src_p[perm]
    dst_s = dst_p[perm]
    bounds = jnp.searchsorted(
        dst_s,
        jnp.arange(NHALVES * NW + 1, dtype=jnp.int32) * RPT).astype(jnp.int32)

    row2 = lambda v: v.reshape(1, -1)
    h = _fuse(x_text, x_vis, tp_w.T, vp_w.T, row2(tp_b + vp_b),
              row2(fln_g), row2(fln_b))

    layers = [
        (l0_msg_w, l0_self_w, l0_self_b, l0_ln_g, l0_ln_b),
        (l1_msg_w, l1_self_w, l1_self_b, l1_ln_g, l1_ln_b),
        (l2_msg_w, l2_self_w, l2_self_b, l2_ln_g, l2_ln_b),
    ]
    for (mw, sw, sb, lg, lb) in layers:
        parts = [
            _sc_segment_sum(h, src_s, dst_s,
                            bounds[hf * NW:(hf + 1) * NW],
                            bounds[hf * NW + 1:(hf + 1) * NW + 1], hf)
            for hf in range(NHALVES)
        ]
        s = jnp.concatenate([p[0] for p in parts])
        dg = jnp.concatenate([p[1] for p in parts])
        h = _layer(h, s, dg, mw.T, sw.T, row2(sb), row2(lg), row2(lb))

    batch_r = batch.reshape(NB, 1, BN)
    h2w_pad = jnp.zeros((128, H), jnp.float32).at[0].set(h2_w[0])
    logits = _pool(batch_r, h, h1_w.T, row2(h1_b), h2w_pad, h2_b.reshape(1, 1))
    return logits[:, 0]


# speedup vs baseline: 1.1562x; 1.1562x over previous
import jax
import jax.numpy as jnp
from jax import lax
from jax.experimental import pallas as pl
from jax.experimental.pallas import tpu as pltpu
from jax.experimental.pallas import tpu_sc as plsc

N = 10000
E = 160000
D = 256
H = 256
G = 64

NCORES = 2
NSUB = 16
NW = NCORES * NSUB

NPAD = 10240
RPT = NPAD // NW
ACCR = RPT + 8
EPAD2 = E + 256

CHUNK = 48
LG = 16

BN = 400
NB = N // BN


def _pick(vec, i):
    s = jnp.int32(0)
    for lane in range(LG):
        s = jnp.where(i == lane, vec[lane], s)
    return s


def _tile_bounds(sc, t, blo_hbm, bhi_hbm, bnd_v):
    pltpu.sync_copy(blo_hbm, bnd_v)
    blo_s = _pick(bnd_v[pl.ds(sc * LG, LG)], t)
    pltpu.sync_copy(bhi_hbm, bnd_v)
    bhi_s = _pick(bnd_v[pl.ds(sc * LG, LG)], t)
    clo = (blo_s >> 3) << 3
    nch = (bhi_s - clo + (CHUNK - 1)) // CHUNK
    return clo, nch


def _sc_body(h_hbm, src_hbm, dst_hbm, blo_hbm, bhi_hbm, s_out,
             sidx0, sidx1, didx0, didx1, rows0, rows1, bnd_v, acc_v,
             gsem, ssem0, ssem1, dsem0, dsem1):
    sc = lax.axis_index("c")
    t = lax.axis_index("s")
    wid = sc * NSUB + t
    zf = jnp.zeros((LG,), jnp.float32)
    sidx = (sidx0, sidx1)
    didx = (didx0, didx1)
    rows = (rows0, rows1)
    ssem = (ssem0, ssem1)
    dsem = (dsem0, dsem1)

    def zero_body(r, carry):
        for j in range(H // LG):
            acc_v[r, pl.ds(LG * j, LG)] = zf
        return carry

    lax.fori_loop(0, ACCR, zero_body, 0)

    clo, nch = _tile_bounds(sc, t, blo_hbm, bhi_hbm, bnd_v)
    nch2 = ((nch + 1) >> 1) << 1
    lo_node = wid * RPT

    def e_at(c):
        return pl.multiple_of(clo + c * CHUNK, 8)

    def start_idx(c, b):
        pltpu.async_copy(src_hbm.at[pl.ds(e_at(c), CHUNK)], sidx[b], ssem[b])
        pltpu.async_copy(dst_hbm.at[pl.ds(e_at(c), CHUNK)], didx[b], dsem[b])

    def wait_idx(b):
        pltpu.make_async_copy(src_hbm.at[pl.ds(0, CHUNK)], sidx[b],
                              ssem[b]).wait()
        pltpu.make_async_copy(dst_hbm.at[pl.ds(0, CHUNK)], didx[b],
                              dsem[b]).wait()

    def start_gather(b):
        pltpu.async_copy(h_hbm.at[sidx[b]], rows[b], gsem)

    def wait_gather(b):
        pltpu.make_async_copy(h_hbm.at[sidx[b]], rows[b], gsem).wait()

    def accum(b):
        for g in range(CHUNK // LG):
            d = didx[b][pl.ds(g * LG, LG)]
            ld = d - lo_node
            ld = jnp.where((ld >= 0) & (ld < RPT), ld, RPT)
            for lane in range(LG):
                ld_s = ld[lane]
                e = g * LG + lane
                for j in range(H // LG):
                    v = rows[b][e, pl.ds(LG * j, LG)]
                    plsc.addupdate(acc_v.at[ld_s, pl.ds(LG * j, LG)], v)

    start_idx(0, 0)
    wait_idx(0)
    start_gather(0)
    start_idx(1, 1)

    def pair_body(cc, carry):
        for b in range(2):
            c = cc * 2 + b
            wait_gather(b)
            wait_idx(1 - b)
            start_gather(1 - b)

            @pl.when(c < nch)
            def _():
                accum(b)
            start_idx(c + 2, b)
        return carry

    lax.fori_loop(0, nch2 >> 1, pair_body, 0)
    wait_gather(0)
    wait_idx(1)

    out_r = wid * RPT
    pltpu.sync_copy(acc_v.at[pl.ds(0, RPT)], s_out.at[pl.ds(out_r, RPT)])


def _sc_segment_sum(h, src_s, dst_s, blo, bhi):
    mesh = plsc.VectorSubcoreMesh(core_axis_name="c", subcore_axis_name="s")
    k = pl.kernel(
        _sc_body,
        out_type=[jax.ShapeDtypeStruct((NPAD, H), jnp.float32)],
        mesh=mesh,
        scratch_types=[
            pltpu.VMEM((CHUNK,), jnp.int32),
            pltpu.VMEM((CHUNK,), jnp.int32),
            pltpu.VMEM((CHUNK,), jnp.int32),
            pltpu.VMEM((CHUNK,), jnp.int32),
            pltpu.VMEM((CHUNK, H), jnp.float32),
            pltpu.VMEM((CHUNK, H), jnp.float32),
            pltpu.VMEM((NW,), jnp.int32),
            pltpu.VMEM((ACCR, H), jnp.float32),
            pltpu.SemaphoreType.DMA,
            pltpu.SemaphoreType.DMA,
            pltpu.SemaphoreType.DMA,
            pltpu.SemaphoreType.DMA,
            pltpu.SemaphoreType.DMA,
        ],
    )
    return k(h, src_s, dst_s, blo, bhi)


def _sc_deg_body(dst_hbm, blo_hbm, bhi_hbm, deg_out, didx_v, bnd_v, dacc_v):
    sc = lax.axis_index("c")
    t = lax.axis_index("s")
    wid = sc * NSUB + t
    zf = jnp.zeros((LG,), jnp.float32)
    ones = jnp.full((LG,), 1.0, jnp.float32)

    def zero_body(r, carry):
        dacc_v[r] = zf
        return carry

    lax.fori_loop(0, ACCR, zero_body, 0)

    clo, nch = _tile_bounds(sc, t, blo_hbm, bhi_hbm, bnd_v)
    lo_node = wid * RPT

    def chunk_body(c, carry):
        e0 = pl.multiple_of(clo + c * CHUNK, 8)
        pltpu.sync_copy(dst_hbm.at[pl.ds(e0, CHUNK)], didx_v)
        for g in range(CHUNK // LG):
            d = didx_v[pl.ds(g * LG, LG)]
            ld = d - lo_node
            ld = jnp.where((ld >= 0) & (ld < RPT), ld, RPT)
            for lane in range(LG):
                plsc.addupdate(dacc_v.at[ld[lane]], ones)
        return carry

    lax.fori_loop(0, nch, chunk_body, 0)
    out_r = wid * RPT
    pltpu.sync_copy(dacc_v.at[pl.ds(0, RPT)], deg_out.at[pl.ds(out_r, RPT)])


def _sc_degree(dst_s, blo, bhi):
    mesh = plsc.VectorSubcoreMesh(core_axis_name="c", subcore_axis_name="s")
    k = pl.kernel(
        _sc_deg_body,
        out_type=[jax.ShapeDtypeStruct((NPAD, LG), jnp.float32)],
        mesh=mesh,
        scratch_types=[
            pltpu.VMEM((CHUNK,), jnp.int32),
            pltpu.VMEM((NW,), jnp.int32),
            pltpu.VMEM((ACCR, LG), jnp.float32),
        ],
    )
    return k(dst_s, blo, bhi)


def _ln(acc, g, b):
    mu = jnp.mean(acc, axis=-1, keepdims=True)
    var = jnp.mean((acc - mu) ** 2, axis=-1, keepdims=True)
    return (acc - mu) * lax.rsqrt(var + 1e-5) * g + b


def _fuse_body(xt_ref, xv_ref, twt_ref, vwt_ref, b_ref, g_ref, bb_ref, o_ref):
    acc = lax.dot_general(xt_ref[...], twt_ref[...], (((1,), (0,)), ((), ())),
                          preferred_element_type=jnp.float32)
    acc = acc + lax.dot_general(xv_ref[...], vwt_ref[...],
                                (((1,), (0,)), ((), ())),
                                preferred_element_type=jnp.float32)
    acc = acc + b_ref[...]
    o_ref[...] = _ln(acc, g_ref[...], bb_ref[...])


def _fuse(xt, xv, twt, vwt, b, g, bb):
    return pl.pallas_call(
        _fuse_body,
        grid=(NB,),
        in_specs=[
            pl.BlockSpec((BN, D), lambda i: (i, 0)),
            pl.BlockSpec((BN, D), lambda i: (i, 0)),
            pl.BlockSpec((D, H), lambda i: (0, 0)),
            pl.BlockSpec((D, H), lambda i: (0, 0)),
            pl.BlockSpec((1, H), lambda i: (0, 0)),
            pl.BlockSpec((1, H), lambda i: (0, 0)),
            pl.BlockSpec((1, H), lambda i: (0, 0)),
        ],
        out_specs=pl.BlockSpec((BN, H), lambda i: (i, 0)),
        out_shape=jax.ShapeDtypeStruct((N, H), jnp.float32),
    )(xt, xv, twt, vwt, b, g, bb)


def _layer_body(h_ref, s_ref, d_ref, mwt_ref, swt_ref,
                b_ref, g_ref, bb_ref, o_ref):
    h = h_ref[...]
    dinv = 1.0 / jnp.maximum(d_ref[...][:, 0:1], 1.0)
    agg = s_ref[...] * dinv
    acc = lax.dot_general(h, swt_ref[...], (((1,), (0,)), ((), ())),
                          preferred_element_type=jnp.float32)
    acc = acc + lax.dot_general(agg, mwt_ref[...], (((1,), (0,)), ((), ())),
                                preferred_element_type=jnp.float32)
    acc = jnp.maximum(acc + b_ref[...], 0.0)
    o_ref[...] = h + _ln(acc, g_ref[...], bb_ref[...])


def _layer(h, s, d, mwt, swt, b, g, bb):
    return pl.pallas_call(
        _layer_body,
        grid=(NB,),
        in_specs=[
            pl.BlockSpec((BN, H), lambda i: (i, 0)),
            pl.BlockSpec((BN, H), lambda i: (i, 0)),
            pl.BlockSpec((BN, LG), lambda i: (i, 0)),
            pl.BlockSpec((H, H), lambda i: (0, 0)),
            pl.BlockSpec((H, H), lambda i: (0, 0)),
            pl.BlockSpec((1, H), lambda i: (0, 0)),
            pl.BlockSpec((1, H), lambda i: (0, 0)),
            pl.BlockSpec((1, H), lambda i: (0, 0)),
        ],
        out_specs=pl.BlockSpec((BN, H), lambda i: (i, 0)),
        out_shape=jax.ShapeDtypeStruct((N, H), jnp.float32),
    )(h, s, d, mwt, swt, b, g, bb)


def _pool_body(batch_ref, h_ref, h1wt_ref, h1b_ref, h2w_ref, h2b_ref, o_ref,
               sums_acc, cnt_acc):
    i = pl.program_id(0)

    @pl.when(i == 0)
    def _():
        sums_acc[...] = jnp.zeros_like(sums_acc)
        cnt_acc[...] = jnp.zeros_like(cnt_acc)

    b = batch_ref[0]
    gid = lax.broadcasted_iota(jnp.int32, (G, BN), 0)
    onehot = (gid == jnp.broadcast_to(b, (G, BN))).astype(jnp.float32)
    sums_acc[...] += lax.dot_general(onehot, h_ref[...],
                                     (((1,), (0,)), ((), ())),
                                     preferred_element_type=jnp.float32)
    cnt_acc[...] += lax.dot_general(onehot, jnp.ones((BN, H), jnp.float32),
                                    (((1,), (0,)), ((), ())),
                                    preferred_element_type=jnp.float32)

    @pl.when(i == NB - 1)
    def _():
        gmean = sums_acc[...] * (1.0 / jnp.maximum(cnt_acc[...], 1.0))
        z = lax.dot_general(gmean, h1wt_ref[...], (((1,), (0,)), ((), ())),
                            preferred_element_type=jnp.float32)
        z = jnp.maximum(z + h1b_ref[...], 0.0)
        lg = lax.dot_general(z, h2w_ref[...], (((1,), (1,)), ((), ())),
                             preferred_element_type=jnp.float32)
        o_ref[...] = lg + h2b_ref[0, 0]


def _pool(batch_r, h, h1wt, h1b, h2w, h2b):
    return pl.pallas_call(
        _pool_body,
        grid=(NB,),
        in_specs=[
            pl.BlockSpec((1, 1, BN), lambda i: (i, 0, 0)),
            pl.BlockSpec((BN, H), lambda i: (i, 0)),
            pl.BlockSpec((H, H), lambda i: (0, 0)),
            pl.BlockSpec((1, H), lambda i: (0, 0)),
            pl.BlockSpec((128, H), lambda i: (0, 0)),
            pl.BlockSpec((1, 1), lambda i: (0, 0)),
        ],
        out_specs=pl.BlockSpec((G, 128), lambda i: (0, 0)),
        out_shape=jax.ShapeDtypeStruct((G, 128), jnp.float32),
        scratch_shapes=[
            pltpu.VMEM((G, H), jnp.float32),
            pltpu.VMEM((G, H), jnp.float32),
        ],
    )(batch_r, h, h1wt, h1b, h2w, h2b)


def kernel(x_text, x_vis, tp_w, tp_b, vp_w, vp_b, fln_g, fln_b,
           l0_msg_w, l0_self_w, l0_self_b, l0_ln_g, l0_ln_b,
           l1_msg_w, l1_self_w, l1_self_b, l1_ln_g, l1_ln_b,
           l2_msg_w, l2_self_w, l2_self_b, l2_ln_g, l2_ln_b,
           h1_w, h1_b, h2_w, h2_b, edge_index, batch):
    src = edge_index[0]
    dst = edge_index[1]
    src_p = jnp.concatenate([src, jnp.zeros((EPAD2 - E,), jnp.int32)])
    dst_p = jnp.concatenate([dst, jnp.full((EPAD2 - E,), NPAD, jnp.int32)])
    perm = jnp.argsort(dst_p)
    src_s = src_p[perm]
    dst_s = dst_p[perm]
    bounds = jnp.searchsorted(
        dst_s, jnp.arange(NW + 1, dtype=jnp.int32) * RPT).astype(jnp.int32)
    blo = bounds[:NW]
    bhi = bounds[1:]

    row2 = lambda v: v.reshape(1, -1)
    h = _fuse(x_text, x_vis, tp_w.T, vp_w.T, row2(tp_b + vp_b),
              row2(fln_g), row2(fln_b))

    layers = [
        (l0_msg_w, l0_self_w, l0_self_b, l0_ln_g, l0_ln_b),
        (l1_msg_w, l1_self_w, l1_self_b, l1_ln_g, l1_ln_b),
        (l2_msg_w, l2_self_w, l2_self_b, l2_ln_g, l2_ln_b),
    ]
    dg, = _sc_degree(dst_s, blo, bhi)
    for (mw, sw, sb, lg, lb) in layers:
        s, = _sc_segment_sum(h, src_s, dst_s, blo, bhi)
        h = _layer(h, s, dg, mw.T, sw.T, row2(sb), row2(lg), row2(lb))

    batch_r = batch.reshape(NB, 1, BN)
    h2w_pad = jnp.zeros((128, H), jnp.float32).at[0].set(h2_w[0])
    logits = _pool(batch_r, h, h1_w.T, row2(h1_b), h2w_pad, h2_b.reshape(1, 1))
    return logits[:, 0]


# speedup vs baseline: 1.4796x; 1.2798x over previous
import jax
import jax.numpy as jnp
from jax import lax
from jax.experimental import pallas as pl
from jax.experimental.pallas import tpu as pltpu
from jax.experimental.pallas import tpu_sc as plsc

N = 10000
E = 160000
D = 256
H = 256
G = 64

NCORES = 2
NSUB = 16
NW = NCORES * NSUB

NPAD = 10240
RPT = NPAD // NW
ACCR = RPT + 8
EPAD2 = E + 384

CHUNK = 40
NSLOT = 3
LG = 16

BN = 400
NB = N // BN


def _pick(vec, i):
    s = jnp.int32(0)
    for lane in range(LG):
        s = jnp.where(i == lane, vec[lane], s)
    return s


def _tile_bounds(sc, t, blo_hbm, bhi_hbm, bnd_v):
    pltpu.sync_copy(blo_hbm, bnd_v)
    blo_s = _pick(bnd_v[pl.ds(sc * LG, LG)], t)
    pltpu.sync_copy(bhi_hbm, bnd_v)
    bhi_s = _pick(bnd_v[pl.ds(sc * LG, LG)], t)
    clo = (blo_s >> 3) << 3
    nch = (bhi_s - clo + (CHUNK - 1)) // CHUNK
    return clo, nch


def _sc_body(h_hbm, src_hbm, dst_hbm, blo_hbm, bhi_hbm, s_out,
             sidx0, sidx1, sidx2, didx0, didx1, didx2, rows0, rows1, rows2,
             bnd_v, acc_v,
             gsem, ssem0, ssem1, ssem2, dsem0, dsem1, dsem2):
    sc = lax.axis_index("c")
    t = lax.axis_index("s")
    wid = sc * NSUB + t
    zf = jnp.zeros((LG,), jnp.float32)
    sidx = (sidx0, sidx1, sidx2)
    didx = (didx0, didx1, didx2)
    rows = (rows0, rows1, rows2)
    ssem = (ssem0, ssem1, ssem2)
    dsem = (dsem0, dsem1, dsem2)

    def zero_body(r, carry):
        for j in range(H // LG):
            acc_v[r, pl.ds(LG * j, LG)] = zf
        return carry

    lax.fori_loop(0, ACCR, zero_body, 0)

    clo, nch = _tile_bounds(sc, t, blo_hbm, bhi_hbm, bnd_v)
    nch3 = ((nch + 2) // 3) * 3
    lo_node = wid * RPT

    def e_at(c):
        return pl.multiple_of(clo + c * CHUNK, 8)

    def start_idx(c, b):
        pltpu.async_copy(src_hbm.at[pl.ds(e_at(c), CHUNK)], sidx[b], ssem[b])
        pltpu.async_copy(dst_hbm.at[pl.ds(e_at(c), CHUNK)], didx[b], dsem[b])

    def wait_idx(b):
        pltpu.make_async_copy(src_hbm.at[pl.ds(0, CHUNK)], sidx[b],
                              ssem[b]).wait()
        pltpu.make_async_copy(dst_hbm.at[pl.ds(0, CHUNK)], didx[b],
                              dsem[b]).wait()

    def start_gather(b):
        pltpu.async_copy(h_hbm.at[sidx[b]], rows[b], gsem)

    def wait_gather(b):
        pltpu.make_async_copy(h_hbm.at[sidx[b]], rows[b], gsem).wait()

    def accum(b):
        for g in range(CHUNK // LG):
            d = didx[b][pl.ds(g * LG, LG)]
            ld = d - lo_node
            ld = jnp.where((ld >= 0) & (ld < RPT), ld, RPT)
            for lane in range(LG):
                ld_s = ld[lane]
                e = g * LG + lane
                for j in range(H // LG):
                    v = rows[b][e, pl.ds(LG * j, LG)]
                    plsc.addupdate(acc_v.at[ld_s, pl.ds(LG * j, LG)], v)

    start_idx(0, 0)
    start_idx(1, 1)
    start_idx(2, 2)
    wait_idx(0)
    start_gather(0)
    wait_idx(1)
    start_gather(1)

    def trip_body(cc, carry):
        for b in range(NSLOT):
            c = cc * NSLOT + b

            wait_gather(b)

            @pl.when(c < nch)
            def _():
                accum(b)
            start_idx(c + NSLOT, b)
            wait_idx((b + 2) % NSLOT)
            start_gather((b + 2) % NSLOT)
        return carry

    lax.fori_loop(0, nch3 // NSLOT, trip_body, 0)
    wait_gather(0)
    wait_gather(1)
    wait_idx(2)

    out_r = wid * RPT
    pltpu.sync_copy(acc_v.at[pl.ds(0, RPT)], s_out.at[pl.ds(out_r, RPT)])


def _sc_segment_sum(h, src_s, dst_s, blo, bhi):
    mesh = plsc.VectorSubcoreMesh(core_axis_name="c", subcore_axis_name="s")
    k = pl.kernel(
        _sc_body,
        out_type=[jax.ShapeDtypeStruct((NPAD, H), jnp.float32)],
        mesh=mesh,
        scratch_types=[
            pltpu.VMEM((CHUNK,), jnp.int32),
            pltpu.VMEM((CHUNK,), jnp.int32),
            pltpu.VMEM((CHUNK,), jnp.int32),
            pltpu.VMEM((CHUNK,), jnp.int32),
            pltpu.VMEM((CHUNK,), jnp.int32),
            pltpu.VMEM((CHUNK,), jnp.int32),
            pltpu.VMEM((CHUNK, H), jnp.float32),
            pltpu.VMEM((CHUNK, H), jnp.float32),
            pltpu.VMEM((CHUNK, H), jnp.float32),
            pltpu.VMEM((NW,), jnp.int32),
            pltpu.VMEM((ACCR, H), jnp.float32),
            pltpu.SemaphoreType.DMA,
            pltpu.SemaphoreType.DMA,
            pltpu.SemaphoreType.DMA,
            pltpu.SemaphoreType.DMA,
            pltpu.SemaphoreType.DMA,
            pltpu.SemaphoreType.DMA,
            pltpu.SemaphoreType.DMA,
        ],
    )
    return k(h, src_s, dst_s, blo, bhi)


def _sc_deg_body(dst_hbm, blo_hbm, bhi_hbm, deg_out, didx_v, bnd_v, dacc_v):
    sc = lax.axis_index("c")
    t = lax.axis_index("s")
    wid = sc * NSUB + t
    zf = jnp.zeros((LG,), jnp.float32)
    ones = jnp.full((LG,), 1.0, jnp.float32)

    def zero_body(r, carry):
        dacc_v[r] = zf
        return carry

    lax.fori_loop(0, ACCR, zero_body, 0)

    clo, nch = _tile_bounds(sc, t, blo_hbm, bhi_hbm, bnd_v)
    lo_node = wid * RPT

    def chunk_body(c, carry):
        e0 = pl.multiple_of(clo + c * CHUNK, 8)
        pltpu.sync_copy(dst_hbm.at[pl.ds(e0, CHUNK)], didx_v)
        for g in range(CHUNK // LG):
            d = didx_v[pl.ds(g * LG, LG)]
            ld = d - lo_node
            ld = jnp.where((ld >= 0) & (ld < RPT), ld, RPT)
            for lane in range(LG):
                plsc.addupdate(dacc_v.at[ld[lane]], ones)
        return carry

    lax.fori_loop(0, nch, chunk_body, 0)
    out_r = wid * RPT
    pltpu.sync_copy(dacc_v.at[pl.ds(0, RPT)], deg_out.at[pl.ds(out_r, RPT)])


def _sc_degree(dst_s, blo, bhi):
    mesh = plsc.VectorSubcoreMesh(core_axis_name="c", subcore_axis_name="s")
    k = pl.kernel(
        _sc_deg_body,
        out_type=[jax.ShapeDtypeStruct((NPAD, LG), jnp.float32)],
        mesh=mesh,
        scratch_types=[
            pltpu.VMEM((CHUNK,), jnp.int32),
            pltpu.VMEM((NW,), jnp.int32),
            pltpu.VMEM((ACCR, LG), jnp.float32),
        ],
    )
    return k(dst_s, blo, bhi)


def _ln(acc, g, b):
    mu = jnp.mean(acc, axis=-1, keepdims=True)
    var = jnp.mean((acc - mu) ** 2, axis=-1, keepdims=True)
    return (acc - mu) * lax.rsqrt(var + 1e-5) * g + b


def _fuse_body(xt_ref, xv_ref, twt_ref, vwt_ref, b_ref, g_ref, bb_ref, o_ref):
    acc = lax.dot_general(xt_ref[...], twt_ref[...], (((1,), (0,)), ((), ())),
                          preferred_element_type=jnp.float32)
    acc = acc + lax.dot_general(xv_ref[...], vwt_ref[...],
                                (((1,), (0,)), ((), ())),
                                preferred_element_type=jnp.float32)
    acc = acc + b_ref[...]
    o_ref[...] = _ln(acc, g_ref[...], bb_ref[...])


def _fuse(xt, xv, twt, vwt, b, g, bb):
    return pl.pallas_call(
        _fuse_body,
        grid=(NB,),
        in_specs=[
            pl.BlockSpec((BN, D), lambda i: (i, 0)),
            pl.BlockSpec((BN, D), lambda i: (i, 0)),
            pl.BlockSpec((D, H), lambda i: (0, 0)),
            pl.BlockSpec((D, H), lambda i: (0, 0)),
            pl.BlockSpec((1, H), lambda i: (0, 0)),
            pl.BlockSpec((1, H), lambda i: (0, 0)),
            pl.BlockSpec((1, H), lambda i: (0, 0)),
        ],
        out_specs=pl.BlockSpec((BN, H), lambda i: (i, 0)),
        out_shape=jax.ShapeDtypeStruct((N, H), jnp.float32),
    )(xt, xv, twt, vwt, b, g, bb)


def _layer_body(h_ref, s_ref, d_ref, mwt_ref, swt_ref,
                b_ref, g_ref, bb_ref, o_ref):
    h = h_ref[...]
    dinv = 1.0 / jnp.maximum(d_ref[...][:, 0:1], 1.0)
    agg = s_ref[...] * dinv
    acc = lax.dot_general(h, swt_ref[...], (((1,), (0,)), ((), ())),
                          preferred_element_type=jnp.float32)
    acc = acc + lax.dot_general(agg, mwt_ref[...], (((1,), (0,)), ((), ())),
                                preferred_element_type=jnp.float32)
    acc = jnp.maximum(acc + b_ref[...], 0.0)
    o_ref[...] = h + _ln(acc, g_ref[...], bb_ref[...])


def _layer(h, s, d, mwt, swt, b, g, bb):
    return pl.pallas_call(
        _layer_body,
        grid=(NB,),
        in_specs=[
            pl.BlockSpec((BN, H), lambda i: (i, 0)),
            pl.BlockSpec((BN, H), lambda i: (i, 0)),
            pl.BlockSpec((BN, LG), lambda i: (i, 0)),
            pl.BlockSpec((H, H), lambda i: (0, 0)),
            pl.BlockSpec((H, H), lambda i: (0, 0)),
            pl.BlockSpec((1, H), lambda i: (0, 0)),
            pl.BlockSpec((1, H), lambda i: (0, 0)),
            pl.BlockSpec((1, H), lambda i: (0, 0)),
        ],
        out_specs=pl.BlockSpec((BN, H), lambda i: (i, 0)),
        out_shape=jax.ShapeDtypeStruct((N, H), jnp.float32),
    )(h, s, d, mwt, swt, b, g, bb)


def _pool_body(batch_ref, h_ref, h1wt_ref, h1b_ref, h2w_ref, h2b_ref, o_ref,
               sums_acc, cnt_acc):
    i = pl.program_id(0)

    @pl.when(i == 0)
    def _():
        sums_acc[...] = jnp.zeros_like(sums_acc)
        cnt_acc[...] = jnp.zeros_like(cnt_acc)

    b = batch_ref[0]
    gid = lax.broadcasted_iota(jnp.int32, (G, BN), 0)
    onehot = (gid == jnp.broadcast_to(b, (G, BN))).astype(jnp.float32)
    sums_acc[...] += lax.dot_general(onehot, h_ref[...],
                                     (((1,), (0,)), ((), ())),
                                     preferred_element_type=jnp.float32)
    cnt_acc[...] += lax.dot_general(onehot, jnp.ones((BN, H), jnp.float32),
                                    (((1,), (0,)), ((), ())),
                                    preferred_element_type=jnp.float32)

    @pl.when(i == NB - 1)
    def _():
        gmean = sums_acc[...] * (1.0 / jnp.maximum(cnt_acc[...], 1.0))
        z = lax.dot_general(gmean, h1wt_ref[...], (((1,), (0,)), ((), ())),
                            preferred_element_type=jnp.float32)
        z = jnp.maximum(z + h1b_ref[...], 0.0)
        lg = lax.dot_general(z, h2w_ref[...], (((1,), (1,)), ((), ())),
                             preferred_element_type=jnp.float32)
        o_ref[...] = lg + h2b_ref[0, 0]


def _pool(batch_r, h, h1wt, h1b, h2w, h2b):
    return pl.pallas_call(
        _pool_body,
        grid=(NB,),
        in_specs=[
            pl.BlockSpec((1, 1, BN), lambda i: (i, 0, 0)),
            pl.BlockSpec((BN, H), lambda i: (i, 0)),
            pl.BlockSpec((H, H), lambda i: (0, 0)),
            pl.BlockSpec((1, H), lambda i: (0, 0)),
            pl.BlockSpec((128, H), lambda i: (0, 0)),
            pl.BlockSpec((1, 1), lambda i: (0, 0)),
        ],
        out_specs=pl.BlockSpec((G, 128), lambda i: (0, 0)),
        out_shape=jax.ShapeDtypeStruct((G, 128), jnp.float32),
        scratch_shapes=[
            pltpu.VMEM((G, H), jnp.float32),
            pltpu.VMEM((G, H), jnp.float32),
        ],
    )(batch_r, h, h1wt, h1b, h2w, h2b)


def kernel(x_text, x_vis, tp_w, tp_b, vp_w, vp_b, fln_g, fln_b,
           l0_msg_w, l0_self_w, l0_self_b, l0_ln_g, l0_ln_b,
           l1_msg_w, l1_self_w, l1_self_b, l1_ln_g, l1_ln_b,
           l2_msg_w, l2_self_w, l2_self_b, l2_ln_g, l2_ln_b,
           h1_w, h1_b, h2_w, h2_b, edge_index, batch):
    src = edge_index[0]
    dst = edge_index[1]
    src_p = jnp.concatenate([src, jnp.zeros((EPAD2 - E,), jnp.int32)])
    dst_p = jnp.concatenate([dst, jnp.full((EPAD2 - E,), NPAD, jnp.int32)])
    perm = jnp.argsort(dst_p)
    src_s = src_p[perm]
    dst_s = dst_p[perm]
    bounds = jnp.searchsorted(
        dst_s, jnp.arange(NW + 1, dtype=jnp.int32) * RPT).astype(jnp.int32)
    blo = bounds[:NW]
    bhi = bounds[1:]

    row2 = lambda v: v.reshape(1, -1)
    h = _fuse(x_text, x_vis, tp_w.T, vp_w.T, row2(tp_b + vp_b),
              row2(fln_g), row2(fln_b))

    layers = [
        (l0_msg_w, l0_self_w, l0_self_b, l0_ln_g, l0_ln_b),
        (l1_msg_w, l1_self_w, l1_self_b, l1_ln_g, l1_ln_b),
        (l2_msg_w, l2_self_w, l2_self_b, l2_ln_g, l2_ln_b),
    ]
    dg, = _sc_degree(dst_s, blo, bhi)
    for (mw, sw, sb, lg, lb) in layers:
        s, = _sc_segment_sum(h, src_s, dst_s, blo, bhi)
        h = _layer(h, s, dg, mw.T, sw.T, row2(sb), row2(lg), row2(lb))

    batch_r = batch.reshape(NB, 1, BN)
    h2w_pad = jnp.zeros((128, H), jnp.float32).at[0].set(h2_w[0])
    logits = _pool(batch_r, h, h1_w.T, row2(h1_b), h2w_pad, h2_b.reshape(1, 1))
    return logits[:, 0]
